# baseline (speedup 1.0000x reference)
import functools

import jax
import jax.numpy as jnp
from jax import lax
from jax.experimental import pallas as pl
from jax.experimental.pallas import tpu as pltpu
from jax.experimental.pallas import tpu_sc as plsc

_B = 16384
_D_TAB = 56
_D_OUT = 64


@functools.cache
def _make_sc_kernel():
    info = plsc.get_sparse_core_info()
    nw = info.num_cores * info.num_subcores
    b_per_w = _B // nw
    mesh = plsc.VectorSubcoreMesh(core_axis_name="c", subcore_axis_name="s")

    @functools.partial(
        pl.kernel,
        mesh=mesh,
        out_type=jax.ShapeDtypeStruct((_B, _D_OUT), jnp.float32),
        compiler_params=pltpu.CompilerParams(use_tc_tiling_on_sc=False),
        scratch_types=[
            pltpu.VMEM((b_per_w,), jnp.int32),
            pltpu.VMEM((b_per_w, _D_TAB), jnp.float32),
            pltpu.VMEM((b_per_w, _D_OUT - _D_TAB), jnp.float32),
            pltpu.SemaphoreType.DMA,
        ],
    )
    def k(x_hbm, table_hbm, tail_hbm, out_hbm, idx_v, g_v, tail_v, sem):
        wid = lax.axis_index("s") * info.num_cores + lax.axis_index("c")
        base = wid * b_per_w

        pltpu.sync_copy(x_hbm.at[pl.ds(base, b_per_w)], idx_v)
        pltpu.sync_copy(tail_hbm, tail_v)

        pltpu.async_copy(table_hbm.at[idx_v], g_v, sem).wait()

        pltpu.sync_copy(g_v, out_hbm.at[pl.ds(base, b_per_w), pl.ds(0, _D_TAB)])
        pltpu.sync_copy(
            tail_v, out_hbm.at[pl.ds(base, b_per_w), pl.ds(_D_TAB, _D_OUT - _D_TAB)]
        )

    return k


def kernel(x, table, shared):
    nw_rows = _B // 32
    tail_rows = jnp.broadcast_to(
        jnp.reshape(shared, (1, _D_OUT - _D_TAB)), (nw_rows, _D_OUT - _D_TAB)
    )
    out = _make_sc_kernel()(x.astype(jnp.int32), table, tail_rows)
    return out[:, None, :]


# speedup vs baseline: 1.4925x; 1.4925x over previous
import functools

import jax
import jax.numpy as jnp
from jax import lax
from jax.experimental import pallas as pl
from jax.experimental.pallas import tpu as pltpu
from jax.experimental.pallas import tpu_sc as plsc

_B = 16384
_D_TAB = 56
_D_OUT = 64
_G = 16


@functools.cache
def _make_sc_kernel():
    info = plsc.get_sparse_core_info()
    nw = info.num_cores * info.num_subcores
    b_per_w = _B // nw
    n_groups = b_per_w // _G
    mesh = plsc.VectorSubcoreMesh(core_axis_name="c", subcore_axis_name="s")

    @functools.partial(
        pl.kernel,
        mesh=mesh,
        out_type=jax.ShapeDtypeStruct((_B, _D_OUT), jnp.float32),
        scratch_types=[
            pltpu.VMEM((b_per_w,), jnp.int32),
            pltpu.VMEM((2 * _G * 8, _D_TAB), jnp.float32),
            pltpu.VMEM((b_per_w, _D_OUT), jnp.float32),
            pltpu.VMEM((16,), jnp.float32),
            pltpu.SemaphoreType.DMA,
            pltpu.SemaphoreType.DMA,
        ],
    )
    def k(
        x_hbm, table3_hbm, tail_hbm, out_hbm,
        idx_v, blk_v, rows_v, tail_v, sem0, sem1,
    ):
        sems = (sem0, sem1)
        wid = lax.axis_index("s") * info.num_cores + lax.axis_index("c")
        base = wid * b_per_w

        pltpu.sync_copy(x_hbm.at[pl.ds(base, b_per_w)], idx_v)
        pltpu.sync_copy(tail_hbm, tail_v)
        v_tail = tail_v[...]


        def fire(g, buf):
            xv = idx_v[pl.ds(g * _G, _G)]
            t_vec = xv >> 3
            for i in range(_G):
                t_i = lax.squeeze(lax.slice(t_vec, (i,), (i + 1,)), (0,))
                pltpu.async_copy(
                    table3_hbm.at[t_i],
                    blk_v.at[pl.ds((buf * _G + i) * 8, 8)],
                    sems[buf],
                )

        def drain(buf):
            pltpu.make_async_copy(
                table3_hbm.at[pl.ds(0, _G)],
                blk_v.at[pl.ds(buf * _G * 8, _G * 8)],
                sems[buf],
            ).wait()

        def extract(g, buf):
            xv = idx_v[pl.ds(g * _G, _G)]
            r_vec = xv & 7
            for i in range(_G):
                r_i = lax.squeeze(lax.slice(r_vec, (i,), (i + 1,)), (0,))
                rs = (buf * _G + i) * 8 + r_i
                row = g * _G + i
                rows_v[row, pl.ds(48, 16)] = v_tail
                rows_v[row, pl.ds(0, 16)] = blk_v[rs, pl.ds(0, 16)]
                rows_v[row, pl.ds(16, 16)] = blk_v[rs, pl.ds(16, 16)]
                rows_v[row, pl.ds(32, 16)] = blk_v[rs, pl.ds(32, 16)]
                rows_v[row, pl.ds(40, 16)] = blk_v[rs, pl.ds(40, 16)]

        fire(0, 0)

        def body(go, c):
            g0 = go * 2
            g1 = g0 + 1
            fire(g1, 1)
            drain(0)
            extract(g0, 0)

            @pl.when(go < n_groups // 2 - 1)
            def _():
                fire(g0 + 2, 0)

            drain(1)
            extract(g1, 1)
            return c

        lax.fori_loop(0, n_groups // 2, body, 0)

        pltpu.sync_copy(rows_v, out_hbm.at[pl.ds(base, b_per_w)])

    return k


def kernel(x, table, shared):
    table3 = jnp.reshape(table, (125000, 8, _D_TAB))
    tail16 = jnp.tile(jnp.reshape(shared, (_D_OUT - _D_TAB,)), 2)
    out = _make_sc_kernel()(x.astype(jnp.int32), table3, tail16)
    return out[:, None, :]


# speedup vs baseline: 3.7289x; 2.4984x over previous
import functools

import jax
import jax.numpy as jnp
from jax import lax
from jax.experimental import pallas as pl
from jax.experimental.pallas import tpu as pltpu
from jax.experimental.pallas import tpu_sc as plsc

_B = 16384
_D_TAB = 56
_D_OUT = 64
_G = 16


@functools.cache
def _make_sc_kernel():
    info = plsc.get_sparse_core_info()
    nw = info.num_cores * info.num_subcores
    b_per_w = _B // nw
    n_groups = b_per_w // _G
    mesh = plsc.VectorSubcoreMesh(core_axis_name="c", subcore_axis_name="s")

    @functools.partial(
        pl.kernel,
        mesh=mesh,
        out_type=jax.ShapeDtypeStruct((_B, _D_OUT), jnp.float32),
        scratch_types=[
            pltpu.VMEM((b_per_w,), jnp.int32),
            pltpu.VMEM((2 * _G * 8, _D_TAB), jnp.float32),
            pltpu.VMEM((b_per_w, _D_OUT), jnp.float32),
            pltpu.VMEM((16,), jnp.float32),
            pltpu.SemaphoreType.DMA,
            pltpu.SemaphoreType.DMA,
        ],
    )
    def k(
        x_hbm, table_hbm, tail_hbm, out_hbm,
        idx_v, blk_v, rows_v, tail_v, sem0, sem1,
    ):
        sems = (sem0, sem1)
        wid = lax.axis_index("s") * info.num_cores + lax.axis_index("c")
        base = wid * b_per_w

        pltpu.sync_copy(x_hbm.at[pl.ds(base, b_per_w)], idx_v)
        pltpu.sync_copy(tail_hbm, tail_v)
        v_tail = tail_v[...]


        def fire(g, buf):
            xv = idx_v[pl.ds(g * _G, _G)]
            t_vec = xv >> 3
            for i in range(_G):
                t_i = lax.squeeze(lax.slice(t_vec, (i,), (i + 1,)), (0,))
                start = pl.multiple_of(t_i * 8, 8)
                pltpu.async_copy(
                    table_hbm.at[pl.ds(start, 8)],
                    blk_v.at[pl.ds((buf * _G + i) * 8, 8)],
                    sems[buf],
                )

        def drain(buf):
            pltpu.make_async_copy(
                table_hbm.at[pl.ds(0, _G * 8)],
                blk_v.at[pl.ds(buf * _G * 8, _G * 8)],
                sems[buf],
            ).wait()

        def extract(g, buf):
            xv = idx_v[pl.ds(g * _G, _G)]
            r_vec = xv & 7
            for i in range(_G):
                r_i = lax.squeeze(lax.slice(r_vec, (i,), (i + 1,)), (0,))
                rs = (buf * _G + i) * 8 + r_i
                row = g * _G + i
                rows_v[row, pl.ds(48, 16)] = v_tail
                rows_v[row, pl.ds(0, 16)] = blk_v[rs, pl.ds(0, 16)]
                rows_v[row, pl.ds(16, 16)] = blk_v[rs, pl.ds(16, 16)]
                rows_v[row, pl.ds(32, 16)] = blk_v[rs, pl.ds(32, 16)]
                rows_v[row, pl.ds(40, 16)] = blk_v[rs, pl.ds(40, 16)]

        fire(0, 0)

        def body(go, c):
            g0 = go * 2
            g1 = g0 + 1
            fire(g1, 1)
            drain(0)
            extract(g0, 0)

            @pl.when(go < n_groups // 2 - 1)
            def _():
                fire(g0 + 2, 0)

            drain(1)
            extract(g1, 1)
            return c

        lax.fori_loop(0, n_groups // 2, body, 0)

        pltpu.sync_copy(rows_v, out_hbm.at[pl.ds(base, b_per_w)])

    return k


def kernel(x, table, shared):
    tail16 = jnp.tile(jnp.reshape(shared, (_D_OUT - _D_TAB,)), 2)
    out = _make_sc_kernel()(x.astype(jnp.int32), table, tail16)
    return out[:, None, :]
